# Initial kernel scaffold; baseline (speedup 1.0000x reference)
import jax
import jax.numpy as jnp
from jax.experimental import pallas as pl


def kernel(points, point_id_offset, lidar_features, image_features, Wq, bq, Wk, bk, Wv, bv, Wc, bc):
    raise NotImplementedError("write your pallas kernel here")


# speedup vs baseline: 13.3335x; 13.3335x over previous
import jax
import jax.numpy as jnp
from jax.experimental import pallas as pl
from jax.experimental.pallas import tpu as pltpu

HIDDEN_C = 256
K_NEIGH = 16
ROW_TILE = 512

_DN_TT = (((1,), (1,)), ((), ()))


def _bf16_dot_t(a, b):
    return jax.lax.dot_general(
        a.astype(jnp.bfloat16), b.astype(jnp.bfloat16), _DN_TT,
        preferred_element_type=jnp.float32)


def _fusion_kernel(pts_full_ref, pts_tile_ref, lidar_tile_ref, image_ref,
                   wq_ref, bq_ref, wk_ref, bk_ref, wv_ref, bv_ref,
                   wc_ref, bc_ref, out_ref, k_scr, v_scr, stat_scr):
    t = pl.program_id(1)
    pts_full = pts_full_ref[0]
    pts_t = pts_tile_ref[0]

    @pl.when(t == 0)
    def _():
        image = image_ref[0]
        k_scr[...] = (_bf16_dot_t(image, wk_ref[...])
                      + bk_ref[0]).astype(jnp.bfloat16)
        v_scr[...] = (_bf16_dot_t(image, wv_ref[...])
                      + bv_ref[0]).astype(jnp.bfloat16)
        psq = pts_full * pts_full
        ones_3 = jnp.ones((1, 3), jnp.float32)
        stat_scr[0:1, :] = jax.lax.dot_general(
            ones_3, psq, _DN_TT, preferred_element_type=jnp.float32,
            precision=jax.lax.Precision.HIGHEST)
        ones_c = jnp.ones((1, image.shape[1]), jnp.float32)
        stat_scr[1:2, :] = jax.lax.dot_general(
            ones_c, image, _DN_TT, preferred_element_type=jnp.float32,
            precision=jax.lax.Precision.HIGHEST)

    sq_row = stat_scr[0:1, :]
    valid_row = stat_scr[1:2, :] != 0.0

    sq_t = jnp.sum(pts_t * pts_t, axis=1, keepdims=True)
    d2 = (sq_t + sq_row) - 2.0 * _bf16_dot_t(pts_t, pts_full)

    inf = jnp.float32(jnp.inf)

    def extract(_, carry):
        cur, _m = carry
        m = jnp.min(cur, axis=1, keepdims=True)
        cur = jnp.where(cur <= m, inf, cur)
        return cur, m

    _, thresh = jax.lax.fori_loop(
        0, K_NEIGH, extract,
        (d2, jnp.zeros((d2.shape[0], 1), jnp.float32)))
    mask = (d2 <= thresh) & valid_row

    scale = 1.0 / jnp.sqrt(jnp.float32(HIDDEN_C))
    q_t = _bf16_dot_t(lidar_tile_ref[0], wq_ref[...]) + bq_ref[0]
    s = _bf16_dot_t(q_t, k_scr[...]) * scale
    s = jnp.where(mask, s, -inf)
    mx = jnp.max(s, axis=1, keepdims=True)
    mx = jnp.where(mx == -inf, 0.0, mx)
    p = jnp.where(mask, jnp.exp(s - mx), 0.0)
    denom = jnp.sum(p, axis=1, keepdims=True)
    attn = p / jnp.where(denom == 0.0, 1.0, denom)

    o = jax.lax.dot_general(
        attn.astype(jnp.bfloat16), v_scr[...],
        (((1,), (0,)), ((), ())), preferred_element_type=jnp.float32)
    out_ref[0] = _bf16_dot_t(o, wc_ref[...]) + bc_ref[0]


def kernel(points, point_id_offset, lidar_features, image_features,
           Wq, bq, Wk, bk, Wv, bv, Wc, bc):
    b = point_id_offset.shape[0]
    n = points.shape[0]
    npts = n // b
    c = lidar_features.shape[1]
    n_tiles = npts // ROW_TILE

    pts = points.reshape(b, npts, 3)
    lidar = lidar_features.reshape(b, npts, c)
    image = image_features.reshape(b, npts, c)
    bq2 = bq.reshape(1, -1)
    bk2 = bk.reshape(1, -1)
    bv2 = bv.reshape(1, -1)
    bc2 = bc.reshape(1, -1)

    full_spec = lambda shape: pl.BlockSpec(shape, lambda i, j: (0, 0))

    out = pl.pallas_call(
        _fusion_kernel,
        grid=(b, n_tiles),
        in_specs=[
            pl.BlockSpec((1, npts, 3), lambda i, j: (i, 0, 0)),
            pl.BlockSpec((1, ROW_TILE, 3), lambda i, j: (i, j, 0)),
            pl.BlockSpec((1, ROW_TILE, c), lambda i, j: (i, j, 0)),
            pl.BlockSpec((1, npts, c), lambda i, j: (i, 0, 0)),
            full_spec(Wq.shape), full_spec(bq2.shape),
            full_spec(Wk.shape), full_spec(bk2.shape),
            full_spec(Wv.shape), full_spec(bv2.shape),
            full_spec(Wc.shape), full_spec(bc2.shape),
        ],
        out_specs=pl.BlockSpec((1, ROW_TILE, c), lambda i, j: (i, j, 0)),
        out_shape=jax.ShapeDtypeStruct((b, npts, c), jnp.float32),
        scratch_shapes=[
            pltpu.VMEM((npts, c), jnp.bfloat16),
            pltpu.VMEM((npts, c), jnp.bfloat16),
            pltpu.VMEM((8, npts), jnp.float32),
        ],
    )(pts, pts, lidar, image, Wq, bq2, Wk, bk2, Wv, bv2, Wc, bc2)
    return out.reshape(n, c)


# speedup vs baseline: 21.9973x; 1.6498x over previous
import jax
import jax.numpy as jnp
from jax.experimental import pallas as pl
from jax.experimental.pallas import tpu as pltpu

HIDDEN_C = 256
K_NEIGH = 16
ROW_TILE = 512

_DN_TT = (((1,), (1,)), ((), ()))


def _bf16_dot_t(a, b):
    return jax.lax.dot_general(
        a.astype(jnp.bfloat16), b.astype(jnp.bfloat16), _DN_TT,
        preferred_element_type=jnp.float32)


def _fusion_kernel(pts_full_ref, pts_tile_ref, lidar_tile_ref, image_ref,
                   wq_ref, bq_ref, wk_ref, bk_ref, wv_ref, bv_ref,
                   wc_ref, bc_ref, out_ref, k_scr, v_scr, stat_scr):
    t = pl.program_id(1)
    pts_full = pts_full_ref[0]
    pts_t = pts_tile_ref[0]

    @pl.when(t == 0)
    def _():
        image = image_ref[0]
        k_scr[...] = (_bf16_dot_t(image, wk_ref[...])
                      + bk_ref[0]).astype(jnp.bfloat16)
        v_scr[...] = (_bf16_dot_t(image, wv_ref[...])
                      + bv_ref[0]).astype(jnp.bfloat16)
        psq = pts_full * pts_full
        ones_3 = jnp.ones((1, 3), jnp.float32)
        stat_scr[0:1, :] = jax.lax.dot_general(
            ones_3, psq, _DN_TT, preferred_element_type=jnp.float32,
            precision=jax.lax.Precision.HIGHEST)
        ones_c = jnp.ones((1, image.shape[1]), jnp.float32)
        stat_scr[1:2, :] = jax.lax.dot_general(
            ones_c, image, _DN_TT, preferred_element_type=jnp.float32,
            precision=jax.lax.Precision.HIGHEST)

    sq_row = stat_scr[0:1, :]
    valid_row = stat_scr[1:2, :] != 0.0

    sq_t = jnp.sum(pts_t * pts_t, axis=1, keepdims=True)
    d2 = (sq_t + sq_row) - 2.0 * _bf16_dot_t(pts_t, pts_full)

    inf = jnp.float32(jnp.inf)
    rows = d2.shape[0]

    def extract(_, carry):
        cur, _m = carry
        m = jnp.min(cur, axis=1, keepdims=True)
        cur = jnp.where(cur <= m, inf, cur)
        return cur, m

    chunks = [d2[:, g * 128:(g + 1) * 128] for g in range(d2.shape[1] // 128)]
    cand = []
    for i in range(4):
        m = chunks[0]
        for p in chunks[1:]:
            m = jnp.minimum(m, p)
        cand.append(m)
        if i < 3:
            chunks = [jnp.where(p <= m, inf, p) for p in chunks]
    cand_mat = jnp.concatenate(cand, axis=1)

    _, thresh = jax.lax.fori_loop(
        0, K_NEIGH, extract,
        (cand_mat, jnp.zeros((rows, 1), jnp.float32)))

    counts = jnp.sum((d2 <= thresh).astype(jnp.float32), axis=1,
                     keepdims=True)
    ok = jnp.max(counts) == jnp.float32(K_NEIGH)

    def _exact_thresh():
        _, th = jax.lax.fori_loop(
            0, K_NEIGH, extract,
            (d2, jnp.zeros((rows, 1), jnp.float32)))
        return th

    thresh = jax.lax.cond(ok, lambda: thresh, _exact_thresh)
    mask = (d2 <= thresh) & valid_row

    scale = 1.0 / jnp.sqrt(jnp.float32(HIDDEN_C))
    q_t = _bf16_dot_t(lidar_tile_ref[0], wq_ref[...]) + bq_ref[0]
    s = _bf16_dot_t(q_t, k_scr[...]) * scale
    s = jnp.where(mask, s, -inf)
    mx = jnp.max(s, axis=1, keepdims=True)
    mx = jnp.where(mx == -inf, 0.0, mx)
    p = jnp.where(mask, jnp.exp(s - mx), 0.0)
    denom = jnp.sum(p, axis=1, keepdims=True)
    attn = p / jnp.where(denom == 0.0, 1.0, denom)

    o = jax.lax.dot_general(
        attn.astype(jnp.bfloat16), v_scr[...],
        (((1,), (0,)), ((), ())), preferred_element_type=jnp.float32)
    out_ref[0] = _bf16_dot_t(o, wc_ref[...]) + bc_ref[0]


def kernel(points, point_id_offset, lidar_features, image_features,
           Wq, bq, Wk, bk, Wv, bv, Wc, bc):
    b = point_id_offset.shape[0]
    n = points.shape[0]
    npts = n // b
    c = lidar_features.shape[1]
    n_tiles = npts // ROW_TILE

    pts = points.reshape(b, npts, 3)
    lidar = lidar_features.reshape(b, npts, c)
    image = image_features.reshape(b, npts, c)
    bq2 = bq.reshape(1, -1)
    bk2 = bk.reshape(1, -1)
    bv2 = bv.reshape(1, -1)
    bc2 = bc.reshape(1, -1)

    full_spec = lambda shape: pl.BlockSpec(shape, lambda i, j: (0, 0))

    out = pl.pallas_call(
        _fusion_kernel,
        grid=(b, n_tiles),
        in_specs=[
            pl.BlockSpec((1, npts, 3), lambda i, j: (i, 0, 0)),
            pl.BlockSpec((1, ROW_TILE, 3), lambda i, j: (i, j, 0)),
            pl.BlockSpec((1, ROW_TILE, c), lambda i, j: (i, j, 0)),
            pl.BlockSpec((1, npts, c), lambda i, j: (i, 0, 0)),
            full_spec(Wq.shape), full_spec(bq2.shape),
            full_spec(Wk.shape), full_spec(bk2.shape),
            full_spec(Wv.shape), full_spec(bv2.shape),
            full_spec(Wc.shape), full_spec(bc2.shape),
        ],
        out_specs=pl.BlockSpec((1, ROW_TILE, c), lambda i, j: (i, j, 0)),
        out_shape=jax.ShapeDtypeStruct((b, npts, c), jnp.float32),
        scratch_shapes=[
            pltpu.VMEM((npts, c), jnp.bfloat16),
            pltpu.VMEM((npts, c), jnp.bfloat16),
            pltpu.VMEM((8, npts), jnp.float32),
        ],
    )(pts, pts, lidar, image, Wq, bq2, Wk, bk2, Wv, bv2, Wc, bc2)
    return out.reshape(n, c)


# speedup vs baseline: 24.4507x; 1.1115x over previous
import jax
import jax.numpy as jnp
from jax.experimental import pallas as pl
from jax.experimental.pallas import tpu as pltpu

HIDDEN_C = 256
K_NEIGH = 16
ROW_TILE = 512

_DN_TT = (((1,), (1,)), ((), ()))


def _bf16_dot_t(a, b):
    return jax.lax.dot_general(
        a.astype(jnp.bfloat16), b.astype(jnp.bfloat16), _DN_TT,
        preferred_element_type=jnp.float32)


def _fusion_kernel(pts_full_ref, pts_tile_ref, lidar_tile_ref, image_ref,
                   wq_ref, bq_ref, wk_ref, bk_ref, wv_ref, bv_ref,
                   wc_ref, bc_ref, out_ref, k_scr, v_scr, stat_scr):
    t = pl.program_id(1)
    pts_full = pts_full_ref[0]
    pts_t = pts_tile_ref[0]

    @pl.when(t == 0)
    def _():
        image = image_ref[0]
        k_scr[...] = (_bf16_dot_t(image, wk_ref[...])
                      + bk_ref[0]).astype(jnp.bfloat16)
        v_scr[...] = (_bf16_dot_t(image, wv_ref[...])
                      + bv_ref[0]).astype(jnp.bfloat16)
        psq = pts_full * pts_full
        ones_3 = jnp.ones((1, 3), jnp.float32)
        stat_scr[0:1, :] = jax.lax.dot_general(
            ones_3, psq, _DN_TT, preferred_element_type=jnp.float32,
            precision=jax.lax.Precision.HIGHEST)
        ones_c = jnp.ones((1, image.shape[1]), jnp.float32)
        stat_scr[1:2, :] = jax.lax.dot_general(
            ones_c, image, _DN_TT, preferred_element_type=jnp.float32,
            precision=jax.lax.Precision.HIGHEST)

    sq_row = stat_scr[0:1, :]
    valid_row = stat_scr[1:2, :] != 0.0

    sq_t = jnp.sum(pts_t * pts_t, axis=1, keepdims=True)
    d2 = (sq_t + sq_row) - 2.0 * _bf16_dot_t(pts_t, pts_full)

    inf = jnp.float32(jnp.inf)
    rows = d2.shape[0]

    def extract(_, carry):
        cur, _m = carry
        m = jnp.min(cur, axis=1, keepdims=True)
        cur = jnp.where(cur <= m, inf, cur)
        return cur, m

    def ce(a, b):
        return jnp.minimum(a, b), jnp.maximum(a, b)

    def bottom4(a, b):
        c = [jnp.minimum(a[i], b[3 - i]) for i in range(4)]
        l0, h0 = ce(c[0], c[2])
        l1, h1 = ce(c[1], c[3])
        m0, m1 = ce(l0, l1)
        m2, m3 = ce(h0, h1)
        return [m0, m1, m2, m3]

    chunks = [d2[:, g * 128:(g + 1) * 128] for g in range(d2.shape[1] // 128)]
    s4 = []
    for g in range(4):
        a0, a1 = ce(chunks[4 * g], chunks[4 * g + 1])
        b0, b1 = ce(chunks[4 * g + 2], chunks[4 * g + 3])
        c0, t0 = ce(a0, b0)
        t1, c3 = ce(a1, b1)
        c1, c2 = ce(t0, t1)
        s4.append([c0, c1, c2, c3])
    heads = bottom4(bottom4(s4[0], s4[1]), bottom4(s4[2], s4[3]))

    def extract_heads(_, carry):
        s0, s1, s2, s3, _m = carry
        m = jnp.min(s0, axis=1, keepdims=True)
        eq = s0 <= m
        return (jnp.where(eq, s1, s0), jnp.where(eq, s2, s1),
                jnp.where(eq, s3, s2), jnp.where(eq, inf, s3), m)

    *_, thresh = jax.lax.fori_loop(
        0, K_NEIGH, extract_heads,
        (heads[0], heads[1], heads[2], heads[3],
         jnp.zeros((rows, 1), jnp.float32)))

    counts = jnp.sum((d2 <= thresh).astype(jnp.float32), axis=1,
                     keepdims=True)
    ok = jnp.max(counts) == jnp.float32(K_NEIGH)

    def _exact_thresh():
        _, th = jax.lax.fori_loop(
            0, K_NEIGH, extract,
            (d2, jnp.zeros((rows, 1), jnp.float32)))
        return th

    thresh = jax.lax.cond(ok, lambda: thresh, _exact_thresh)
    mask = (d2 <= thresh) & valid_row

    scale = 1.0 / jnp.sqrt(jnp.float32(HIDDEN_C))
    q_t = (_bf16_dot_t(lidar_tile_ref[0], wq_ref[...]) + bq_ref[0]) * scale
    s = _bf16_dot_t(q_t, k_scr[...])
    s = jnp.where(mask, s, -inf)
    mx = jnp.max(s, axis=1, keepdims=True)
    mx = jnp.where(mx == -inf, 0.0, mx)
    p = jnp.exp(s - mx)
    denom = jnp.sum(p, axis=1, keepdims=True)

    attn = p / jnp.where(denom == 0.0, 1.0, denom)
    o = jax.lax.dot_general(
        attn.astype(jnp.bfloat16), v_scr[...],
        (((1,), (0,)), ((), ())), preferred_element_type=jnp.float32)
    out_ref[0] = _bf16_dot_t(o, wc_ref[...]) + bc_ref[0]


def kernel(points, point_id_offset, lidar_features, image_features,
           Wq, bq, Wk, bk, Wv, bv, Wc, bc):
    b = point_id_offset.shape[0]
    n = points.shape[0]
    npts = n // b
    c = lidar_features.shape[1]
    n_tiles = npts // ROW_TILE

    pts = points.reshape(b, npts, 3)
    lidar = lidar_features.reshape(b, npts, c)
    image = image_features.reshape(b, npts, c)
    bq2 = bq.reshape(1, -1)
    bk2 = bk.reshape(1, -1)
    bv2 = bv.reshape(1, -1)
    bc2 = bc.reshape(1, -1)

    full_spec = lambda shape: pl.BlockSpec(shape, lambda i, j: (0, 0))

    out = pl.pallas_call(
        _fusion_kernel,
        grid=(b, n_tiles),
        in_specs=[
            pl.BlockSpec((1, npts, 3), lambda i, j: (i, 0, 0)),
            pl.BlockSpec((1, ROW_TILE, 3), lambda i, j: (i, j, 0)),
            pl.BlockSpec((1, ROW_TILE, c), lambda i, j: (i, j, 0)),
            pl.BlockSpec((1, npts, c), lambda i, j: (i, 0, 0)),
            full_spec(Wq.shape), full_spec(bq2.shape),
            full_spec(Wk.shape), full_spec(bk2.shape),
            full_spec(Wv.shape), full_spec(bv2.shape),
            full_spec(Wc.shape), full_spec(bc2.shape),
        ],
        out_specs=pl.BlockSpec((1, ROW_TILE, c), lambda i, j: (i, j, 0)),
        out_shape=jax.ShapeDtypeStruct((b, npts, c), jnp.float32),
        scratch_shapes=[
            pltpu.VMEM((npts, c), jnp.bfloat16),
            pltpu.VMEM((npts, c), jnp.bfloat16),
            pltpu.VMEM((8, npts), jnp.float32),
        ],
    )(pts, pts, lidar, image, Wq, bq2, Wk, bk2, Wv, bv2, Wc, bc2)
    return out.reshape(n, c)


# speedup vs baseline: 30.0535x; 1.2291x over previous
import jax
import jax.numpy as jnp
from jax.experimental import pallas as pl
from jax.experimental.pallas import tpu as pltpu

HIDDEN_C = 256
K_NEIGH = 16
ROW_TILE = 512

_DN_TT = (((1,), (1,)), ((), ()))


def _bf16_dot_t(a, b):
    return jax.lax.dot_general(
        a.astype(jnp.bfloat16), b.astype(jnp.bfloat16), _DN_TT,
        preferred_element_type=jnp.float32)


def _fusion_kernel(pts_full_ref, pts_tile_ref, lidar_tile_ref, image_ref,
                   wq_ref, bq_ref, wk_ref, bk_ref, wv_ref, bv_ref,
                   wc_ref, bc_ref, out_ref, k_scr, v_scr, stat_scr):
    t = pl.program_id(1)
    pts_full = pts_full_ref[0]
    pts_t = pts_tile_ref[0]

    @pl.when(t == 0)
    def _():
        image = image_ref[0]
        k_scr[...] = (_bf16_dot_t(image, wk_ref[...])
                      + bk_ref[0]).astype(jnp.bfloat16)
        v_scr[...] = (_bf16_dot_t(image, wv_ref[...])
                      + bv_ref[0]).astype(jnp.bfloat16)
        psq = pts_full * pts_full
        ones_3 = jnp.ones((1, 3), jnp.float32)
        stat_scr[0:1, :] = jax.lax.dot_general(
            ones_3, psq, _DN_TT, preferred_element_type=jnp.float32,
            precision=jax.lax.Precision.HIGHEST)
        ones_c = jnp.ones((1, image.shape[1]), jnp.float32)
        stat_scr[1:2, :] = jax.lax.dot_general(
            ones_c, image, _DN_TT, preferred_element_type=jnp.float32,
            precision=jax.lax.Precision.HIGHEST)

    sq_row = stat_scr[0:1, :]
    valid_row = stat_scr[1:2, :] != 0.0

    sq_t = jnp.sum(pts_t * pts_t, axis=1, keepdims=True)
    d2 = (sq_t + sq_row) - 2.0 * _bf16_dot_t(pts_t, pts_full)

    inf = jnp.float32(jnp.inf)
    rows = d2.shape[0]

    def extract(_, carry):
        cur, _m = carry
        m = jnp.min(cur, axis=1, keepdims=True)
        cur = jnp.where(cur <= m, inf, cur)
        return cur, m

    def ce(a, b):
        return jnp.minimum(a, b), jnp.maximum(a, b)

    def bottom4(a, b):
        c = [jnp.minimum(a[i], b[3 - i]) for i in range(4)]
        l0, h0 = ce(c[0], c[2])
        l1, h1 = ce(c[1], c[3])
        m0, m1 = ce(l0, l1)
        m2, m3 = ce(h0, h1)
        return [m0, m1, m2, m3]

    chunks = [d2[:, g * 128:(g + 1) * 128] for g in range(d2.shape[1] // 128)]
    s4 = []
    for g in range(4):
        a0, a1 = ce(chunks[4 * g], chunks[4 * g + 1])
        b0, b1 = ce(chunks[4 * g + 2], chunks[4 * g + 3])
        c0, t0 = ce(a0, b0)
        t1, c3 = ce(a1, b1)
        c1, c2 = ce(t0, t1)
        s4.append([c0, c1, c2, c3])
    heads = bottom4(bottom4(s4[0], s4[1]), bottom4(s4[2], s4[3]))

    sub = 64
    thresh_parts = []
    for r0 in range(0, rows, sub):
        s0 = heads[0][r0:r0 + sub, :]
        s1 = heads[1][r0:r0 + sub, :]
        s2 = heads[2][r0:r0 + sub, :]
        s3 = heads[3][r0:r0 + sub, :]
        m = jnp.min(s0, axis=1, keepdims=True)
        for _ in range(K_NEIGH - 1):
            eq = s0 <= m
            s0 = jnp.where(eq, s1, s0)
            s1 = jnp.where(eq, s2, s1)
            s2 = jnp.where(eq, s3, s2)
            s3 = jnp.where(eq, inf, s3)
            m = jnp.min(s0, axis=1, keepdims=True)
        thresh_parts.append(m)
    thresh = jnp.concatenate(thresh_parts, axis=0)

    counts = jnp.sum((d2 <= thresh).astype(jnp.float32), axis=1,
                     keepdims=True)
    ok = jnp.max(counts) == jnp.float32(K_NEIGH)

    def _exact_thresh():
        _, th = jax.lax.fori_loop(
            0, K_NEIGH, extract,
            (d2, jnp.zeros((rows, 1), jnp.float32)))
        return th

    thresh = jax.lax.cond(ok, lambda: thresh, _exact_thresh)
    mask = (d2 <= thresh) & valid_row

    scale = 1.0 / jnp.sqrt(jnp.float32(HIDDEN_C))
    q_t = (_bf16_dot_t(lidar_tile_ref[0], wq_ref[...]) + bq_ref[0]) * scale
    s = _bf16_dot_t(q_t, k_scr[...])
    s = jnp.where(mask, s, -inf)
    mx = jnp.max(s, axis=1, keepdims=True)
    mx = jnp.where(mx == -inf, 0.0, mx)
    p = jnp.exp(s - mx)
    denom = jnp.sum(p, axis=1, keepdims=True)

    attn = p / jnp.where(denom == 0.0, 1.0, denom)
    o = jax.lax.dot_general(
        attn.astype(jnp.bfloat16), v_scr[...],
        (((1,), (0,)), ((), ())), preferred_element_type=jnp.float32)
    out_ref[0] = _bf16_dot_t(o, wc_ref[...]) + bc_ref[0]


def kernel(points, point_id_offset, lidar_features, image_features,
           Wq, bq, Wk, bk, Wv, bv, Wc, bc):
    b = point_id_offset.shape[0]
    n = points.shape[0]
    npts = n // b
    c = lidar_features.shape[1]
    n_tiles = npts // ROW_TILE

    pts = points.reshape(b, npts, 3)
    lidar = lidar_features.reshape(b, npts, c)
    image = image_features.reshape(b, npts, c)
    bq2 = bq.reshape(1, -1)
    bk2 = bk.reshape(1, -1)
    bv2 = bv.reshape(1, -1)
    bc2 = bc.reshape(1, -1)

    full_spec = lambda shape: pl.BlockSpec(shape, lambda i, j: (0, 0))

    out = pl.pallas_call(
        _fusion_kernel,
        grid=(b, n_tiles),
        in_specs=[
            pl.BlockSpec((1, npts, 3), lambda i, j: (i, 0, 0)),
            pl.BlockSpec((1, ROW_TILE, 3), lambda i, j: (i, j, 0)),
            pl.BlockSpec((1, ROW_TILE, c), lambda i, j: (i, j, 0)),
            pl.BlockSpec((1, npts, c), lambda i, j: (i, 0, 0)),
            full_spec(Wq.shape), full_spec(bq2.shape),
            full_spec(Wk.shape), full_spec(bk2.shape),
            full_spec(Wv.shape), full_spec(bv2.shape),
            full_spec(Wc.shape), full_spec(bc2.shape),
        ],
        out_specs=pl.BlockSpec((1, ROW_TILE, c), lambda i, j: (i, j, 0)),
        out_shape=jax.ShapeDtypeStruct((b, npts, c), jnp.float32),
        scratch_shapes=[
            pltpu.VMEM((npts, c), jnp.bfloat16),
            pltpu.VMEM((npts, c), jnp.bfloat16),
            pltpu.VMEM((8, npts), jnp.float32),
        ],
    )(pts, pts, lidar, image, Wq, bq2, Wk, bk2, Wv, bv2, Wc, bc2)
    return out.reshape(n, c)
